# baseline (speedup 1.0000x reference)
import functools

import jax
import jax.numpy as jnp
from jax import lax
from jax.experimental import pallas as pl
from jax.experimental.pallas import tpu as pltpu
from jax.experimental.pallas import tpu_sc as plsc

D_MODEL = 64
SCALE = float(D_MODEL) ** 0.5

NC = 2
NS = 16
NW = NC * NS

S0 = 4096
S1 = 200
IB = S0 // NW
GRING = 4
ORING = 2
DEPTH = 3
BLK = 4


def _emb_body(xt, table, out, idx_all, gbuf, obuf, gsem, ssem):
    wid = lax.axis_index("s") * NC + lax.axis_index("c")
    col0 = wid * IB
    pltpu.sync_copy(xt.at[:, pl.ds(col0, IB)], idx_all)

    def gather_desc(j, s):
        return pltpu.make_async_copy(
            table.at[idx_all.at[j]], gbuf.at[pl.ds(s * IB, IB)], gsem.at[s]
        )

    def store_desc(j, o):
        return pltpu.make_async_copy(
            obuf.at[pl.ds(o * (D_MODEL // 8), D_MODEL // 8)],
            out.at[j, :, wid],
            ssem.at[o],
        )

    for j in range(DEPTH):
        gather_desc(j, j % GRING).start()

    iotas = [lax.iota(jnp.int32, 16) + (bi * 16) for bi in range(IB // 16)]

    def blk_body(blk, carry):
        for b in range(BLK):
            j = blk * BLK + b
            o = b % ORING
            gather_desc(j, b).wait()

            @pl.when(j >= ORING)
            def _drain():
                store_desc(j - ORING, o).wait()

            gbase = b * IB
            obase = o * (D_MODEL // 8)

            def tr_col(c, cc, _gbase=gbase, _obase=obase):
                cvec = jnp.full((16,), 0, jnp.int32) + c
                chi = _obase + lax.shift_right_logical(c, 3)
                clo = lax.rem(c, 8)
                for bi in range(IB // 16):
                    v = plsc.load_gather(gbuf, [_gbase + iotas[bi], cvec])
                    obuf[chi, clo, pl.ds(bi * 16, 16)] = v * SCALE
                return cc

            lax.fori_loop(0, D_MODEL, tr_col, 0)
            store_desc(j, o).start()

            h = j + DEPTH
            hs = (b + DEPTH) % GRING

            @pl.when(h < S1)
            def _fire():
                gather_desc(h, hs).start()

        return carry

    lax.fori_loop(0, S1 // BLK, blk_body, 0)

    for j in range(S1 - ORING, S1):
        store_desc(j, j % ORING).wait()


@functools.partial(jax.jit, static_argnames=())
def _emb_call(xt, table):
    mesh = plsc.VectorSubcoreMesh(core_axis_name="c", subcore_axis_name="s")
    return pl.kernel(
        _emb_body,
        mesh=mesh,
        out_type=jax.ShapeDtypeStruct((S1, 8, NW, 8, 128), jnp.float32),
        scratch_types=[
            pltpu.VMEM((S1, IB), jnp.int32),
            pltpu.VMEM((GRING * IB, D_MODEL), jnp.float32),
            pltpu.VMEM((ORING * (D_MODEL // 8), 8, 128), jnp.float32),
            pltpu.SemaphoreType.DMA((GRING,)),
            pltpu.SemaphoreType.DMA((ORING,)),
        ],
        compiler_params=pltpu.CompilerParams(
            use_tc_tiling_on_sc=False, needs_layout_passes=False
        ),
    )(xt, table)


def kernel(x, table):
    xt = x.T.astype(jnp.int32)
    out5 = _emb_call(xt, table)
    return out5.transpose(2, 4, 0, 1, 3).reshape(S0, S1, D_MODEL)


# speedup vs baseline: 2.0817x; 2.0817x over previous
import functools

import jax
import jax.numpy as jnp
from jax import lax
from jax.experimental import pallas as pl
from jax.experimental.pallas import tpu as pltpu
from jax.experimental.pallas import tpu_sc as plsc

D_MODEL = 64
DPAD = 128
SCALE = float(D_MODEL) ** 0.5

NC = 2
NS = 16
NW = NC * NS

S0 = 4096
S1 = 200
B = S0 * S1
G = 128
NG = B // G
GPW = NG // NW
RING = 8
DEPTH = 6
BLOCKS = GPW // RING


def _emb_body(x2d, table, out, idx_all, rows, gsem, ssem):
    wid = lax.axis_index("s") * NC + lax.axis_index("c")
    g0 = wid * GPW
    pltpu.sync_copy(x2d.at[pl.ds(g0, GPW)], idx_all)

    def gather_desc(g, s):
        return pltpu.make_async_copy(
            table.at[idx_all.at[g]], rows.at[pl.ds(s * G, G)], gsem.at[s]
        )

    def store_desc(g, s):
        return pltpu.make_async_copy(
            rows.at[pl.ds(s * G, G)],
            out.at[pl.ds((g0 + g) * G, G), pl.ds(0, D_MODEL)],
            ssem.at[s],
        )

    for g in range(DEPTH):
        gather_desc(g, g % RING).start()

    def blk_body(blk, carry):
        for b in range(RING):
            g = blk * RING + b
            h = g + DEPTH
            hs = (b + DEPTH) % RING

            @pl.when(h < GPW)
            def _fire():
                @pl.when(h >= RING)
                def _drain():
                    store_desc(h - RING, hs).wait()

                gather_desc(h, hs).start()

            gather_desc(g, b).wait()
            base = b * G

            def scale_row(i, c, _base=base):
                r = _base + i
                for j4 in range(D_MODEL // 16):
                    sl = (r, pl.ds(j4 * 16, 16))
                    rows[sl] = rows[sl] * SCALE
                return c

            lax.fori_loop(0, G, scale_row, 0, unroll=4)
            store_desc(g, b).start()
        return carry

    lax.fori_loop(0, BLOCKS, blk_body, 0)

    for b in range(RING):
        store_desc(GPW - RING + b, b).wait()


@functools.partial(jax.jit, static_argnames=())
def _emb_call(x2d, table):
    mesh = plsc.VectorSubcoreMesh(core_axis_name="c", subcore_axis_name="s")
    return pl.kernel(
        _emb_body,
        mesh=mesh,
        out_type=jax.ShapeDtypeStruct((B, DPAD), jnp.float32),
        scratch_types=[
            pltpu.VMEM((GPW, G), jnp.int32),
            pltpu.VMEM((RING * G, D_MODEL), jnp.float32),
            pltpu.SemaphoreType.DMA((RING,)),
            pltpu.SemaphoreType.DMA((RING,)),
        ],
        compiler_params=pltpu.CompilerParams(
            use_tc_tiling_on_sc=False, needs_layout_passes=False
        ),
    )(x2d, table)


def kernel(x, table):
    x2d = x.reshape(NG, G).astype(jnp.int32)
    out = _emb_call(x2d, table)
    return out[:, :D_MODEL].reshape(S0, S1, D_MODEL)
